# baseline (speedup 1.0000x reference)
import functools

import jax
import jax.numpy as jnp
from jax import lax
from jax.experimental import pallas as pl
from jax.experimental.pallas import tpu as pltpu
from jax.experimental.pallas import tpu_sc as plsc

B, P, V, D = 16384, 12, 28, 24
N = B * P
NC, NS, L = 2, 16, 16
NW = NC * NS
BT = 128
NBT = B // BT
BT_W = NBT // NW
ROWS_W = BT_W * BT * P
CT = D // 8
SLAB = CT * BT_W * 8 * BT
PCT_BLK = BT_W * 8 * BT
P_STRIDE = CT * 8 * B
CT_STRIDE = NBT * 8 * BT
VB = 32


_mesh = plsc.VectorSubcoreMesh(core_axis_name="c", subcore_axis_name="s")


@functools.partial(
    pl.kernel,
    mesh=_mesh,
    out_type=jax.ShapeDtypeStruct((N * D,), jnp.float32),
    compiler_params=pltpu.CompilerParams(needs_layout_passes=False),
    scratch_types=[
        pltpu.VMEM((ROWS_W,), jnp.int32),
        pltpu.VMEM((2 * L, D), jnp.float32),
        pltpu.VMEM((P, D), jnp.float32),
        pltpu.VMEM((P * D * VB,), jnp.float32),
        pltpu.VMEM((2 * SLAB,), jnp.float32),
        pltpu.SemaphoreType.DMA,
        pltpu.SemaphoreType.DMA,
        pltpu.SemaphoreType.DMA,
    ],
)
def _lookup(x_hbm, emb_hbm, pos_hbm, out_hbm,
            x_v, emb_v, pos_v, comb_v, buf_v, sem0, sem1, semx):
    wid = lax.axis_index("s") * NC + lax.axis_index("c")
    col0 = wid * (BT_W * BT)
    xcps = [
        pltpu.make_async_copy(
            x_hbm.at[p, pl.ds(col0, BT_W * BT)],
            x_v.at[pl.ds(p * (BT_W * BT), BT_W * BT)],
            semx,
        )
        for p in range(P)
    ]
    for cp in xcps:
        cp.start()
    pltpu.sync_copy(emb_hbm, emb_v.at[pl.ds(0, V), :])
    pltpu.sync_copy(pos_hbm, pos_v)

    lanes = lax.iota(jnp.int32, L)
    zeros = lanes * 0
    sems = (sem0, sem1)

    @plsc.parallel_loop(0, P * D, unroll=8)
    def _(pc):
        p = lax.div(pc, D)
        c = lax.rem(pc, D)
        g1 = plsc.load_gather(emb_v, [lanes, zeros + c])
        g2 = plsc.load_gather(emb_v, [lanes + L, zeros + c])
        pv = plsc.load_gather(pos_v, [zeros + p, zeros + c])
        comb_v[pl.ds(pc * VB, L)] = g1 + pv
        comb_v[pl.ds(pc * VB + L, L)] = g2 + pv

    for cp in xcps:
        cp.wait()

    out_base = wid * PCT_BLK

    def slab_copies(p, h):
        return [
            pltpu.make_async_copy(
                buf_v.at[pl.ds(h * SLAB + ct * PCT_BLK, PCT_BLK)],
                out_hbm.at[pl.ds(p * P_STRIDE + ct * CT_STRIDE + out_base,
                                 PCT_BLK)],
                sems[h],
            )
            for ct in range(CT)
        ]

    def fill(p, h):
        pb = p * (D * VB)

        @plsc.parallel_loop(0, BT_W * (BT // L), unroll=8)
        def _(u):
            bt = lax.shift_right_logical(u, 3)
            bv = lax.bitwise_and(u, 7)
            xg = x_v[pl.ds(p * (BT_W * BT) + u * L, L)]
            o0 = h * SLAB + bt * 1024 + bv * L
            for ct in range(CT):
                for ci in range(8):
                    blk = comb_v.at[pl.ds(pb + (ct * 8 + ci) * VB, VB)]
                    val = plsc.load_gather(blk, [xg])
                    buf_v[pl.ds(o0 + ct * PCT_BLK + ci * BT, L)] = val

    def outer(p2, carry):
        for h in range(2):
            p = p2 * 2 + h

            @pl.when(p2 > 0)
            def _():
                for cp in slab_copies(p - 2, h):
                    cp.wait()

            fill(p, h)
            for cp in slab_copies(p, h):
                cp.start()
        return carry

    lax.fori_loop(0, P // 2, outer, 0)
    for h in range(2):
        for cp in slab_copies(P - 2 + h, h):
            cp.wait()


def kernel(x, emb_table, pos_table):
    xf = x.T.astype(jnp.int32)
    out = _lookup(xf, emb_table, pos_table)
    return (out.reshape(P, CT, NBT, 8, BT)
            .transpose(2, 4, 0, 1, 3)
            .reshape(B, P, D))


# speedup vs baseline: 1.0094x; 1.0094x over previous
import functools

import jax
import jax.numpy as jnp
from jax import lax
from jax.experimental import pallas as pl
from jax.experimental.pallas import tpu as pltpu
from jax.experimental.pallas import tpu_sc as plsc

B, P, V, D = 16384, 12, 28, 24
N = B * P
NC, NS, L = 2, 16, 16
NW = NC * NS
BT = 128
NBT = B // BT
BT_W = NBT // NW
ROWS_W = BT_W * BT * P
CT = D // 8
SLAB = CT * BT_W * 8 * BT
PCT_BLK = BT_W * 8 * BT
P_STRIDE = CT * 8 * B
CT_STRIDE = NBT * 8 * BT
VB = 32


_mesh = plsc.VectorSubcoreMesh(core_axis_name="c", subcore_axis_name="s")


@functools.partial(
    pl.kernel,
    mesh=_mesh,
    out_type=jax.ShapeDtypeStruct((N * D,), jnp.float32),
    compiler_params=pltpu.CompilerParams(needs_layout_passes=False),
    scratch_types=[
        pltpu.VMEM((ROWS_W,), jnp.int32),
        pltpu.VMEM((2 * L, D), jnp.float32),
        pltpu.VMEM((P, D), jnp.float32),
        pltpu.VMEM((P * D * VB,), jnp.float32),
        pltpu.VMEM((2 * SLAB,), jnp.float32),
        pltpu.SemaphoreType.DMA,
        pltpu.SemaphoreType.DMA,
        pltpu.SemaphoreType.DMA,
    ],
)
def _lookup(x_hbm, emb_hbm, pos_hbm, out_hbm,
            x_v, emb_v, pos_v, comb_v, buf_v, sem0, sem1, semx):
    wid = lax.axis_index("s") * NC + lax.axis_index("c")
    col0 = wid * (BT_W * BT)
    xcps = [
        pltpu.make_async_copy(
            x_hbm.at[p, pl.ds(col0, BT_W * BT)],
            x_v.at[pl.ds(p * (BT_W * BT), BT_W * BT)],
            semx,
        )
        for p in range(P)
    ]
    for cp in xcps:
        cp.start()
    pltpu.sync_copy(emb_hbm, emb_v.at[pl.ds(0, V), :])
    pltpu.sync_copy(pos_hbm, pos_v)

    lanes = lax.iota(jnp.int32, L)
    zeros = lanes * 0
    sems = (sem0, sem1)

    @plsc.parallel_loop(0, P * D, unroll=4)
    def _(pc):
        p = lax.div(pc, D)
        c = lax.rem(pc, D)
        g1 = plsc.load_gather(emb_v, [lanes, zeros + c])
        g2 = plsc.load_gather(emb_v, [lanes + L, zeros + c])
        pv = plsc.load_gather(pos_v, [zeros + p, zeros + c])
        comb_v[pl.ds(pc * VB, L)] = g1 + pv
        comb_v[pl.ds(pc * VB + L, L)] = g2 + pv

    for cp in xcps:
        cp.wait()

    out_base = wid * PCT_BLK

    def slab_copies(p, h):
        return [
            pltpu.make_async_copy(
                buf_v.at[pl.ds(h * SLAB + ct * PCT_BLK, PCT_BLK)],
                out_hbm.at[pl.ds(p * P_STRIDE + ct * CT_STRIDE + out_base,
                                 PCT_BLK)],
                sems[h],
            )
            for ct in range(CT)
        ]

    def fill(p, h):
        pb = p * (D * VB)

        @plsc.parallel_loop(0, BT_W * (BT // L), unroll=4)
        def _(u):
            bt = lax.shift_right_logical(u, 3)
            bv = lax.bitwise_and(u, 7)
            xg = x_v[pl.ds(p * (BT_W * BT) + u * L, L)]
            o0 = h * SLAB + bt * 1024 + bv * L
            for ct in range(CT):
                for ci in range(8):
                    blk = comb_v.at[pl.ds(pb + (ct * 8 + ci) * VB, VB)]
                    val = plsc.load_gather(blk, [xg])
                    buf_v[pl.ds(o0 + ct * PCT_BLK + ci * BT, L)] = val

    def outer(p2, carry):
        for h in range(2):
            p = p2 * 2 + h

            @pl.when(p2 > 0)
            def _():
                for cp in slab_copies(p - 2, h):
                    cp.wait()

            fill(p, h)
            for cp in slab_copies(p, h):
                cp.start()
        return carry

    lax.fori_loop(0, P // 2, outer, 0)
    for h in range(2):
        for cp in slab_copies(P - 2 + h, h):
            cp.wait()


def kernel(x, emb_table, pos_table):
    xf = x.T.astype(jnp.int32)
    out = _lookup(xf, emb_table, pos_table)
    return (out.reshape(P, CT, NBT, 8, BT)
            .transpose(2, 4, 0, 1, 3)
            .reshape(B, P, D))


# speedup vs baseline: 1.1375x; 1.1269x over previous
import functools

import jax
import jax.numpy as jnp
from jax import lax
from jax.experimental import pallas as pl
from jax.experimental.pallas import tpu as pltpu
from jax.experimental.pallas import tpu_sc as plsc

B, P, V, D = 16384, 12, 28, 24
N = B * P
NC, NS, L = 2, 16, 16
NW = NC * NS
BT = 128
NBT = B // BT
BT_W = NBT // NW
ROWS_W = BT_W * BT * P
CT = D // 8
SLAB = CT * BT_W * 8 * BT
PCT_BLK = BT_W * 8 * BT
P_STRIDE = CT * 8 * B
CT_STRIDE = NBT * 8 * BT
VB = 32


_mesh = plsc.VectorSubcoreMesh(core_axis_name="c", subcore_axis_name="s")


@functools.partial(
    pl.kernel,
    mesh=_mesh,
    out_type=jax.ShapeDtypeStruct((N * D,), jnp.float32),
    compiler_params=pltpu.CompilerParams(needs_layout_passes=False),
    scratch_types=[
        pltpu.VMEM((ROWS_W,), jnp.int32),
        pltpu.VMEM((768,), jnp.float32),
        pltpu.VMEM((P * D,), jnp.float32),
        pltpu.VMEM((P * D * VB,), jnp.float32),
        pltpu.VMEM((2 * SLAB,), jnp.float32),
        pltpu.SemaphoreType.DMA,
        pltpu.SemaphoreType.DMA,
        pltpu.SemaphoreType.DMA,
    ],
)
def _lookup(x_hbm, emb_hbm, pos_hbm, out_hbm,
            x_v, emb_v, pos_v, comb_v, buf_v, sem0, sem1, semx):
    wid = lax.axis_index("s") * NC + lax.axis_index("c")
    col0 = wid * (BT_W * BT)
    xcps = [
        pltpu.make_async_copy(
            x_hbm.at[p, pl.ds(col0, BT_W * BT)],
            x_v.at[pl.ds(p * (BT_W * BT), BT_W * BT)],
            semx,
        )
        for p in range(P)
    ]
    for cp in xcps:
        cp.start()
    pltpu.sync_copy(emb_hbm, emb_v.at[pl.ds(0, V * D)])
    pltpu.sync_copy(pos_hbm, pos_v)

    lanes = lax.iota(jnp.int32, L)
    iota24 = lanes * D
    zeros = lanes * 0
    sems = (sem0, sem1)

    @plsc.parallel_loop(0, P * D, unroll=4)
    def _(pc):
        c = lax.rem(pc, D)
        g1 = plsc.load_gather(emb_v, [iota24 + c])
        g2 = plsc.load_gather(emb_v, [iota24 + (c + L * D)])
        pv = plsc.load_gather(pos_v, [zeros + pc])
        comb_v[pl.ds(pc * VB, L)] = g1 + pv
        comb_v[pl.ds(pc * VB + L, L)] = g2 + pv

    for cp in xcps:
        cp.wait()

    out_base = wid * PCT_BLK

    def slab_copies(p, h):
        return [
            pltpu.make_async_copy(
                buf_v.at[pl.ds(h * SLAB + ct * PCT_BLK, PCT_BLK)],
                out_hbm.at[pl.ds(p * P_STRIDE + ct * CT_STRIDE + out_base,
                                 PCT_BLK)],
                sems[h],
            )
            for ct in range(CT)
        ]

    def fill(p, h):
        pb = p * (D * VB)

        @plsc.parallel_loop(0, BT_W * (BT // L), unroll=4)
        def _(u):
            bt = lax.shift_right_logical(u, 3)
            bv = lax.bitwise_and(u, 7)
            xg = x_v[pl.ds(p * (BT_W * BT) + u * L, L)]
            o0 = h * SLAB + bt * 1024 + bv * L
            for ct in range(CT):
                for ci in range(8):
                    blk = comb_v.at[pl.ds(pb + (ct * 8 + ci) * VB, VB)]
                    val = plsc.load_gather(blk, [xg])
                    buf_v[pl.ds(o0 + ct * PCT_BLK + ci * BT, L)] = val

    def outer(p2, carry):
        for h in range(2):
            p = p2 * 2 + h

            @pl.when(p2 > 0)
            def _():
                for cp in slab_copies(p - 2, h):
                    cp.wait()

            fill(p, h)
            for cp in slab_copies(p, h):
                cp.start()
        return carry

    lax.fori_loop(0, P // 2, outer, 0)
    for h in range(2):
        for cp in slab_copies(P - 2 + h, h):
            cp.wait()


def kernel(x, emb_table, pos_table):
    xf = x.T.astype(jnp.int32)
    out = _lookup(xf, emb_table.reshape(-1), pos_table.reshape(-1))
    return (out.reshape(P, CT, NBT, 8, BT)
            .transpose(2, 4, 0, 1, 3)
            .reshape(B, P, D))
